# baseline (speedup 1.0000x reference)
import functools

import jax
import jax.numpy as jnp
from jax import lax
from jax.experimental import pallas as pl
from jax.experimental.pallas import tpu as pltpu
from jax.experimental.pallas import tpu_sc as plsc

NUM_EMBEDDING = 1024
EMBEDDING_DIM = 64
COMMITMENT = 1.0

BT = 4096
SC_CHUNKS = 2


def _tc_body(x_ref, cb_ref, dist_ref, enc_ref, sum_ref):
    pid = pl.program_id(0)
    x = x_ref[...]
    c = cb_ref[...]
    xc = lax.dot_general(x, c, (((1,), (1,)), ((), ())),
                         preferred_element_type=jnp.float32)
    x2 = jnp.sum(x * x, axis=1, keepdims=True)
    c2 = jnp.sum(c * c, axis=1)[None, :]
    dist = x2 + c2 - 2.0 * xc
    dist_ref[...] = dist

    mind = jnp.min(dist, axis=1)
    k_iota = lax.broadcasted_iota(jnp.int32, dist.shape, 1)
    enc = jnp.min(jnp.where(dist == mind[:, None], k_iota, NUM_EMBEDDING),
                  axis=1)
    enc_ref[...] = enc

    acc = sum_ref[...]
    prev = jnp.where(pid == 0, jnp.zeros_like(acc), acc)
    sum_ref[...] = prev + jnp.sum(mind)


def _tc_body_alias(x_ref, cb_ref, dist_in_ref, dist_ref, enc_ref, sum_ref):
    del dist_in_ref
    _tc_body(x_ref, cb_ref, dist_ref, enc_ref, sum_ref)


def _tc_chunk(flat, codebook, dist_prev, base_blocks, nblocks):
    n_tok = flat.shape[0]
    x_spec = pl.BlockSpec((BT, EMBEDDING_DIM),
                          lambda i: (i + base_blocks, 0))
    cb_spec = pl.BlockSpec((NUM_EMBEDDING, EMBEDDING_DIM), lambda i: (0, 0))
    out_specs = [
        pl.BlockSpec((BT, NUM_EMBEDDING), lambda i: (i + base_blocks, 0)),
        pl.BlockSpec((BT,), lambda i: (i,)),
        pl.BlockSpec((1, 1), lambda i: (0, 0)),
    ]
    out_shape = [
        jax.ShapeDtypeStruct((n_tok, NUM_EMBEDDING), jnp.float32),
        jax.ShapeDtypeStruct((nblocks * BT,), jnp.int32),
        jax.ShapeDtypeStruct((1, 1), jnp.float32),
    ]
    if dist_prev is None:
        return pl.pallas_call(
            _tc_body,
            grid=(nblocks,),
            in_specs=[x_spec, cb_spec],
            out_specs=out_specs,
            out_shape=out_shape,
        )(flat, codebook)
    return pl.pallas_call(
        _tc_body_alias,
        grid=(nblocks,),
        in_specs=[x_spec, cb_spec, pl.BlockSpec(memory_space=pl.ANY)],
        out_specs=out_specs,
        out_shape=out_shape,
        input_output_aliases={2: 0},
    )(flat, codebook, dist_prev)


def _sc_gather(codebook, enc, out_shape3):
    info = plsc.get_sparse_core_info()
    nc, ns = info.num_cores, info.num_subcores
    nw = nc * ns
    n_gather = enc.shape[0]
    b_per_w = n_gather // nw
    seq = out_shape3[1]
    rows_per_w = b_per_w // seq
    mesh = plsc.VectorSubcoreMesh(core_axis_name="c", subcore_axis_name="s")

    @functools.partial(
        pl.kernel,
        mesh=mesh,
        compiler_params=pltpu.CompilerParams(use_tc_tiling_on_sc=False),
        out_type=jax.ShapeDtypeStruct(out_shape3, jnp.float32),
        scratch_types=[
            pltpu.VMEM((b_per_w,), jnp.int32),
            pltpu.VMEM((b_per_w, EMBEDDING_DIM), jnp.float32),
            pltpu.SemaphoreType.DMA,
        ],
    )
    def gather_k(table_hbm, idx_hbm, out_hbm, idx_v, rows_v, sem):
        wid = lax.axis_index("s") * nc + lax.axis_index("c")
        base = wid * b_per_w
        pltpu.sync_copy(idx_hbm.at[pl.ds(base, b_per_w)], idx_v)
        pltpu.async_copy(table_hbm.at[idx_v], rows_v, sem).wait()
        if rows_per_w >= 1:
            b0 = base // seq
            pltpu.sync_copy(
                rows_v.reshape(rows_per_w, seq, EMBEDDING_DIM),
                out_hbm.at[pl.ds(b0, rows_per_w)])
        else:
            b0, r0 = base // seq, base % seq
            pltpu.sync_copy(rows_v, out_hbm.at[b0, pl.ds(r0, b_per_w)])

    return gather_k(codebook, enc)


def kernel(input, codebook):
    x = input
    flat = x.reshape(-1, EMBEDDING_DIM)
    n_tok = flat.shape[0]
    nblocks = n_tok // BT

    dist, enc, s = _tc_chunk(flat, codebook, None, 0, nblocks)

    per_tok = n_tok // SC_CHUNKS
    per_batch = x.shape[0] // SC_CHUNKS
    qs = []
    for j in range(SC_CHUNKS):
        enc_j = lax.slice_in_dim(enc, j * per_tok, (j + 1) * per_tok)
        if qs:
            enc_j = lax.optimization_barrier((enc_j, qs[-1]))[0]
        qs.append(_sc_gather(codebook, enc_j,
                             (per_batch,) + x.shape[1:]))

    loss = s[0, 0] * ((1.0 + COMMITMENT) / (n_tok * EMBEDDING_DIM))
    quantized = jnp.concatenate(qs) if SC_CHUNKS > 1 else qs[0]
    quantized_st = quantized
    return (quantized_st, enc, dist, loss)


# speedup vs baseline: 1.0769x; 1.0769x over previous
import functools

import jax
import jax.numpy as jnp
from jax import lax
from jax.experimental import pallas as pl
from jax.experimental.pallas import tpu as pltpu
from jax.experimental.pallas import tpu_sc as plsc

NUM_EMBEDDING = 1024
EMBEDDING_DIM = 64
COMMITMENT = 1.0

BT = 4096
SC_CHUNKS = 1


def _tc_body(x_ref, cb_ref, dist_ref, enc_ref, sum_ref):
    pid = pl.program_id(0)
    x = x_ref[...]
    c = cb_ref[...]
    xc = lax.dot_general(x, c, (((1,), (1,)), ((), ())),
                         preferred_element_type=jnp.float32)
    x2 = jnp.sum(x * x, axis=1, keepdims=True)
    c2 = jnp.sum(c * c, axis=1)[None, :]
    dist = x2 + c2 - 2.0 * xc
    dist_ref[...] = dist

    mind = jnp.min(dist, axis=1)
    k_iota = lax.broadcasted_iota(jnp.int32, dist.shape, 1)
    enc = jnp.min(jnp.where(dist == mind[:, None], k_iota, NUM_EMBEDDING),
                  axis=1)
    enc_ref[...] = enc

    acc = sum_ref[...]
    prev = jnp.where(pid == 0, jnp.zeros_like(acc), acc)
    sum_ref[...] = prev + jnp.sum(mind)


def _tc_body_alias(x_ref, cb_ref, dist_in_ref, dist_ref, enc_ref, sum_ref):
    del dist_in_ref
    _tc_body(x_ref, cb_ref, dist_ref, enc_ref, sum_ref)


def _tc_chunk(flat, codebook, dist_prev, base_blocks, nblocks):
    n_tok = flat.shape[0]
    x_spec = pl.BlockSpec((BT, EMBEDDING_DIM),
                          lambda i: (i + base_blocks, 0))
    cb_spec = pl.BlockSpec((NUM_EMBEDDING, EMBEDDING_DIM), lambda i: (0, 0))
    out_specs = [
        pl.BlockSpec((BT, NUM_EMBEDDING), lambda i: (i + base_blocks, 0)),
        pl.BlockSpec((BT,), lambda i: (i,)),
        pl.BlockSpec((1, 1), lambda i: (0, 0)),
    ]
    out_shape = [
        jax.ShapeDtypeStruct((n_tok, NUM_EMBEDDING), jnp.float32),
        jax.ShapeDtypeStruct((nblocks * BT,), jnp.int32),
        jax.ShapeDtypeStruct((1, 1), jnp.float32),
    ]
    if dist_prev is None:
        return pl.pallas_call(
            _tc_body,
            grid=(nblocks,),
            in_specs=[x_spec, cb_spec],
            out_specs=out_specs,
            out_shape=out_shape,
        )(flat, codebook)
    return pl.pallas_call(
        _tc_body_alias,
        grid=(nblocks,),
        in_specs=[x_spec, cb_spec, pl.BlockSpec(memory_space=pl.ANY)],
        out_specs=out_specs,
        out_shape=out_shape,
        input_output_aliases={2: 0},
    )(flat, codebook, dist_prev)


def _sc_gather(codebook, enc, out_shape3):
    info = plsc.get_sparse_core_info()
    nc, ns = info.num_cores, info.num_subcores
    nw = nc * ns
    n_gather = enc.shape[0]
    b_per_w = n_gather // nw
    seq = out_shape3[1]
    rows_per_w = b_per_w // seq
    mesh = plsc.VectorSubcoreMesh(core_axis_name="c", subcore_axis_name="s")

    @functools.partial(
        pl.kernel,
        mesh=mesh,
        compiler_params=pltpu.CompilerParams(use_tc_tiling_on_sc=False),
        out_type=jax.ShapeDtypeStruct(out_shape3, jnp.float32),
        scratch_types=[
            pltpu.VMEM((b_per_w,), jnp.int32),
            pltpu.VMEM((b_per_w, EMBEDDING_DIM), jnp.float32),
            pltpu.SemaphoreType.DMA,
        ],
    )
    def gather_k(table_hbm, idx_hbm, out_hbm, idx_v, rows_v, sem):
        wid = lax.axis_index("s") * nc + lax.axis_index("c")
        base = wid * b_per_w
        pltpu.sync_copy(idx_hbm.at[pl.ds(base, b_per_w)], idx_v)
        pltpu.async_copy(table_hbm.at[idx_v], rows_v, sem).wait()
        if rows_per_w >= 1:
            b0 = base // seq
            pltpu.sync_copy(
                rows_v.reshape(rows_per_w, seq, EMBEDDING_DIM),
                out_hbm.at[pl.ds(b0, rows_per_w)])
        else:
            b0, r0 = base // seq, base % seq
            pltpu.sync_copy(rows_v, out_hbm.at[b0, pl.ds(r0, b_per_w)])

    return gather_k(codebook, enc)


def kernel(input, codebook):
    x = input
    flat = x.reshape(-1, EMBEDDING_DIM)
    n_tok = flat.shape[0]
    nblocks = n_tok // BT

    dist, enc, s = _tc_chunk(flat, codebook, None, 0, nblocks)

    per_tok = n_tok // SC_CHUNKS
    per_batch = x.shape[0] // SC_CHUNKS
    qs = []
    for j in range(SC_CHUNKS):
        enc_j = lax.slice_in_dim(enc, j * per_tok, (j + 1) * per_tok)
        if qs:
            enc_j = lax.optimization_barrier((enc_j, qs[-1]))[0]
        qs.append(_sc_gather(codebook, enc_j,
                             (per_batch,) + x.shape[1:]))

    loss = s[0, 0] * ((1.0 + COMMITMENT) / (n_tok * EMBEDDING_DIM))
    quantized = jnp.concatenate(qs) if SC_CHUNKS > 1 else qs[0]
    quantized_st = quantized
    return (quantized_st, enc, dist, loss)
